# Initial kernel scaffold; baseline (speedup 1.0000x reference)
import jax
import jax.numpy as jnp
from jax.experimental import pallas as pl


def kernel(x, Wg, bg, W1, b1, W3, b3, W2, b2, Ws1, bs1, Ws3, bs3, Ws2, bs2):
    raise NotImplementedError("write your pallas kernel here")


# speedup vs baseline: 1.9521x; 1.9521x over previous
import functools

import jax
import jax.numpy as jnp
from jax.experimental import pallas as pl

DIM = 1024
INTER = 512
E = 8
TB = 512


def _silu(g):
    return g * (1.0 / (1.0 + jnp.exp(-g)))


def _moe_body(x_ref, wg_ref, bg_ref, w1_ref, b1_ref, w3_ref, b3_ref,
              w2_ref, b2_ref, ws1_ref, bs1_ref, ws3_ref, bs3_ref,
              ws2_ref, bs2_ref, out_ref):
    s = pl.program_id(1)
    x = x_ref[...]
    dn = (((1,), (1,)), ((), ()))
    routed = s < E

    scores = jax.lax.dot_general(x, wg_ref[...], dn,
                                 preferred_element_type=jnp.float32)
    scores = scores + bg_ref[...]
    smax = jnp.max(scores, axis=1, keepdims=True)
    ex = jnp.exp(scores - smax)
    p = ex / jnp.sum(ex, axis=1, keepdims=True)
    iota8 = jax.lax.broadcasted_iota(jnp.int32, (TB, E), 1)
    m1 = jnp.max(p, axis=1, keepdims=True)
    i1 = jnp.min(jnp.where(p == m1, iota8, E), axis=1, keepdims=True)
    pm = jnp.where(iota8 == i1, -jnp.inf, p)
    m2 = jnp.max(pm, axis=1, keepdims=True)
    i2 = jnp.min(jnp.where(pm == m2, iota8, E), axis=1, keepdims=True)
    w_e = jnp.where(i1 == s, m1, 0.0) + jnp.where(i2 == s, m2, 0.0)
    scale = jnp.where(routed, w_e, 1.0)

    A = jnp.where(routed, w1_ref[0], ws1_ref[...])
    B = jnp.where(routed, w3_ref[0], ws3_ref[...])
    C = jnp.where(routed, w2_ref[0], ws2_ref[...])
    ba = jnp.where(routed, b1_ref[0], bs1_ref[...])
    bb = jnp.where(routed, b3_ref[0], bs3_ref[...])
    bc = jnp.where(routed, b2_ref[0], bs2_ref[...])

    g = jax.lax.dot_general(x, A, dn, preferred_element_type=jnp.float32) + ba
    u = jax.lax.dot_general(x, B, dn, preferred_element_type=jnp.float32) + bb
    h = _silu(g) * u
    o = (jax.lax.dot_general(h, C, dn, preferred_element_type=jnp.float32)
         + bc) * scale

    @pl.when(s == 0)
    def _init():
        out_ref[...] = o

    @pl.when(s > 0)
    def _acc():
        out_ref[...] += o


@functools.partial(jax.jit, static_argnames=("interpret",))
def _moe(xf, Wg, bg, W1, b1, W3, b3, W2, b2, Ws1, bs1, Ws3, bs3, Ws2, bs2,
         interpret=False):
    T = xf.shape[0]
    grid = (T // TB, E + 1)
    c2 = lambda shape: pl.BlockSpec(shape, lambda i, s: (0, 0))
    exp3 = lambda shape: pl.BlockSpec(
        (1,) + shape, lambda i, s: (jnp.minimum(s, E - 1), 0, 0))
    exp2 = lambda shape: exp3(shape)
    return pl.pallas_call(
        _moe_body,
        grid=grid,
        in_specs=[
            pl.BlockSpec((TB, DIM), lambda i, s: (i, 0)),
            c2((E, DIM)),
            c2((1, E)),
            exp3((INTER, DIM)),
            exp2((1, INTER)),
            exp3((INTER, DIM)),
            exp2((1, INTER)),
            exp3((DIM, INTER)),
            exp2((1, DIM)),
            c2((INTER, DIM)),
            c2((1, INTER)),
            c2((INTER, DIM)),
            c2((1, INTER)),
            c2((DIM, INTER)),
            c2((1, DIM)),
        ],
        out_specs=pl.BlockSpec((TB, DIM), lambda i, s: (i, 0)),
        out_shape=jax.ShapeDtypeStruct((T, DIM), jnp.float32),
        interpret=interpret,
    )(xf, Wg, bg, W1, b1, W3, b3, W2, b2, Ws1, bs1, Ws3, bs3, Ws2, bs2)


def kernel(x, Wg, bg, W1, b1, W3, b3, W2, b2, Ws1, bs1, Ws3, bs3, Ws2, bs2):
    shape = x.shape
    xf = x.reshape(-1, DIM)
    out = _moe(xf, Wg, bg.reshape(1, E), W1, b1.reshape(E, 1, INTER),
               W3, b3.reshape(E, 1, INTER), W2, b2.reshape(E, 1, DIM),
               Ws1, bs1.reshape(1, INTER), Ws3, bs3.reshape(1, INTER),
               Ws2, bs2.reshape(1, DIM))
    return out.reshape(shape)
